# Initial kernel scaffold; baseline (speedup 1.0000x reference)
import jax
import jax.numpy as jnp
from jax.experimental import pallas as pl


def kernel(codes, pred_class, centroids):
    raise NotImplementedError("write your pallas kernel here")


# speedup vs baseline: 1.1253x; 1.1253x over previous
import jax
import jax.numpy as jnp
from jax import lax
from jax.experimental import pallas as pl
from jax.experimental.pallas import tpu as pltpu
from jax.experimental.pallas import tpu_sc as plsc

NUM_CLASSES = 100000
CODE_DIM = 64
K = 4
BATCH = 16384

_info = plsc.get_sparse_core_info()
_NC, _NS, _L = _info.num_cores, _info.num_subcores, _info.num_lanes
_NW = _NC * _NS
_PW = BATCH // _NW
_CH = 128
_NCHUNK = _PW // _CH
_ROWD = K * CODE_DIM


def _sc_body(codes_hbm, pred_hbm, cents_hbm, out_hbm,
             idx_v, codes_v, cents0, cents1, out_v,
             sem_codes, sem_c0, sem_c1):
    wid = lax.axis_index("s") * _NC + lax.axis_index("c")
    base = wid * _PW

    for c in range(_NCHUNK):
        pltpu.sync_copy(pred_hbm.at[pl.ds(base + c * _CH, _CH)], idx_v.at[c])

    codes_cp = pltpu.async_copy(
        codes_hbm.at[pl.ds(base * CODE_DIM, _PW * CODE_DIM)], codes_v,
        sem_codes)

    cent_bufs = (cents0, cents1)
    sems = (sem_c0, sem_c1)
    cps = [None, None]
    cps[0] = pltpu.async_copy(cents_hbm.at[idx_v.at[0]], cents0, sem_c0)

    codes_cp.wait()
    iota = lax.iota(jnp.int32, _L)
    zero = jnp.zeros((_L,), jnp.float32)

    for c in range(_NCHUNK):
        if c + 1 < _NCHUNK:
            nb = (c + 1) % 2
            cps[nb] = pltpu.async_copy(cents_hbm.at[idx_v.at[c + 1]],
                                       cent_bufs[nb], sems[nb])
        cps[c % 2].wait()
        cbuf = cent_bufs[c % 2]

        def g_body(g, _, c=c, cbuf=cbuf):
            rows = g * _L + iota
            crows = c * _CH + g * _L + iota
            ccols = crows * CODE_DIM
            a0 = a1 = a2 = a3 = zero
            for j in range(CODE_DIM):
                cj = jnp.full((_L,), j, jnp.int32)
                cvec = plsc.load_gather(codes_v, [ccols + j])
                t0 = plsc.load_gather(cbuf, [rows, cj])
                t1 = plsc.load_gather(cbuf, [rows, cj + CODE_DIM])
                t2 = plsc.load_gather(cbuf, [rows, cj + 2 * CODE_DIM])
                t3 = plsc.load_gather(cbuf, [rows, cj + 3 * CODE_DIM])
                a0 = a0 + jnp.abs(cvec - t0)
                a1 = a1 + jnp.abs(cvec - t1)
                a2 = a2 + jnp.abs(cvec - t2)
                a3 = a3 + jnp.abs(cvec - t3)
            m = jnp.minimum(jnp.minimum(a0, a1), jnp.minimum(a2, a3))
            out_v[pl.ds(c * _CH + g * _L, _L)] = m * (1.0 / CODE_DIM)
            return 0

        lax.fori_loop(0, _CH // _L, g_body, 0)

    pltpu.sync_copy(out_v, out_hbm.at[pl.ds(base, _PW)])


_mesh = plsc.VectorSubcoreMesh(core_axis_name="c", subcore_axis_name="s")

_sc_kernel = pl.kernel(
    _sc_body,
    mesh=_mesh,
    out_type=jax.ShapeDtypeStruct((BATCH,), jnp.float32),
    scratch_types=[
        pltpu.VMEM((_NCHUNK, _CH), jnp.int32),
        pltpu.VMEM((_PW * CODE_DIM,), jnp.float32),
        pltpu.VMEM((_CH, _ROWD), jnp.float32),
        pltpu.VMEM((_CH, _ROWD), jnp.float32),
        pltpu.VMEM((_PW,), jnp.float32),
        pltpu.SemaphoreType.DMA,
        pltpu.SemaphoreType.DMA,
        pltpu.SemaphoreType.DMA,
    ],
    compiler_params=pltpu.CompilerParams(needs_layout_passes=False),
)


def kernel(codes, pred_class, centroids):
    pred = pred_class.astype(jnp.int32)
    cents = centroids.reshape(NUM_CLASSES, _ROWD)
    return _sc_kernel(codes.reshape(BATCH * CODE_DIM), pred, cents)


# speedup vs baseline: 1.8240x; 1.6210x over previous
import jax
import jax.numpy as jnp
from jax import lax
from jax.experimental import pallas as pl
from jax.experimental.pallas import tpu as pltpu
from jax.experimental.pallas import tpu_sc as plsc

NUM_CLASSES = 100000
CODE_DIM = 64
K = 4
BATCH = 16384

_info = plsc.get_sparse_core_info()
_NC, _NS, _L = _info.num_cores, _info.num_subcores, _info.num_lanes
_NW = _NC * _NS
_PW = BATCH // _NW
_CH = 128
_NCHUNK = _PW // _CH
_ROWD = K * CODE_DIM


def _sc_body(codes_hbm, pred_hbm, cents_hbm, out_hbm,
             idx_v, codes_v, cents0, cents1, out_v,
             sem_codes, sem_c0, sem_c1):
    wid = lax.axis_index("s") * _NC + lax.axis_index("c")
    base = wid * _PW

    for c in range(_NCHUNK):
        pltpu.sync_copy(pred_hbm.at[pl.ds(base + c * _CH, _CH)], idx_v.at[c])

    codes_cp = pltpu.async_copy(
        codes_hbm.at[pl.ds(base * CODE_DIM, _PW * CODE_DIM)], codes_v,
        sem_codes)

    cent_bufs = (cents0, cents1)
    sems = (sem_c0, sem_c1)
    cps = [None, None]
    cps[0] = pltpu.async_copy(cents_hbm.at[idx_v.at[0]], cents0, sem_c0)

    codes_cp.wait()
    _NV = CODE_DIM // _L
    lane_last = lax.iota(jnp.int32, _L) == (_L - 1)

    for c in range(_NCHUNK):
        if c + 1 < _NCHUNK:
            nb = (c + 1) % 2
            cps[nb] = pltpu.async_copy(cents_hbm.at[idx_v.at[c + 1]],
                                       cent_bufs[nb], sems[nb])
        cps[c % 2].wait()
        cbuf = cent_bufs[c % 2]

        @plsc.parallel_loop(0, _CH, 1, unroll=4)
        def _item(i, c=c, cbuf=cbuf):
            cbase = (c * _CH + i) * CODE_DIM
            code = [codes_v[pl.ds(cbase + v * _L, _L)] for v in range(_NV)]
            s = []
            for k in range(K):
                acc = jnp.abs(code[0] - cbuf[i, pl.ds(k * CODE_DIM, _L)])
                for v in range(1, _NV):
                    t = cbuf[i, pl.ds(k * CODE_DIM + v * _L, _L)]
                    acc = acc + jnp.abs(code[v] - t)
                s.append(plsc.cumsum(acc))
            m = jnp.minimum(jnp.minimum(s[0], s[1]), jnp.minimum(s[2], s[3]))
            m = m * (1.0 / CODE_DIM)
            pos = jnp.full((_L,), c * _CH + i, jnp.int32)
            plsc.store_scatter(out_v, [pos], m, mask=lane_last)

    pltpu.sync_copy(out_v, out_hbm.at[pl.ds(base, _PW)])


_mesh = plsc.VectorSubcoreMesh(core_axis_name="c", subcore_axis_name="s")

_sc_kernel = pl.kernel(
    _sc_body,
    mesh=_mesh,
    out_type=jax.ShapeDtypeStruct((BATCH,), jnp.float32),
    scratch_types=[
        pltpu.VMEM((_NCHUNK, _CH), jnp.int32),
        pltpu.VMEM((_PW * CODE_DIM,), jnp.float32),
        pltpu.VMEM((_CH, _ROWD), jnp.float32),
        pltpu.VMEM((_CH, _ROWD), jnp.float32),
        pltpu.VMEM((_PW,), jnp.float32),
        pltpu.SemaphoreType.DMA,
        pltpu.SemaphoreType.DMA,
        pltpu.SemaphoreType.DMA,
    ],
    compiler_params=pltpu.CompilerParams(needs_layout_passes=False),
)


def kernel(codes, pred_class, centroids):
    pred = pred_class.astype(jnp.int32)
    cents = centroids.reshape(NUM_CLASSES, _ROWD)
    return _sc_kernel(codes.reshape(BATCH * CODE_DIM), pred, cents)
